# Initial kernel scaffold; baseline (speedup 1.0000x reference)
import jax
import jax.numpy as jnp
from jax.experimental import pallas as pl


def kernel(x, W):
    raise NotImplementedError("write your pallas kernel here")


# speedup vs baseline: 2.0838x; 2.0838x over previous
import functools

import jax
import jax.numpy as jnp
from jax import lax
from jax.experimental import pallas as pl
from jax.experimental.pallas import tpu as pltpu
from jax.experimental.pallas import tpu_sc as plsc

TERMS = 1000000
D = 16
NC = 2
NS = 16
NW = NC * NS
GW = 128
K = 16
CHUNK = K * GW


def _sc_gather_clamp(x2, W, n_rows):
    rows_per_w = n_rows // NW
    n_chunks = rows_per_w // K

    mesh = plsc.VectorSubcoreMesh(core_axis_name="c", subcore_axis_name="s")

    @functools.partial(
        pl.kernel,
        mesh=mesh,
        compiler_params=pltpu.CompilerParams(use_tc_tiling_on_sc=False),
        out_type=jax.ShapeDtypeStruct((n_rows * GW, D), jnp.float32),
        scratch_types=[
            pltpu.VMEM((K, GW), jnp.int32),
            pltpu.VMEM((CHUNK, D), jnp.float32),
            pltpu.SemaphoreType.DMA,
        ],
    )
    def k(x_hbm, w_hbm, out_hbm, idx_v, rows_v, sem):
        wid = lax.axis_index("s") * NC + lax.axis_index("c")
        row_base = wid * rows_per_w

        def chunk_body(ci, _):
            row_off = row_base + ci * K
            pltpu.sync_copy(x_hbm.at[pl.ds(row_off, K)], idx_v)
            copies = []
            for j in range(K):
                copies.append(
                    pltpu.async_copy(
                        w_hbm.at[idx_v.at[j]],
                        rows_v.at[pl.ds(j * GW, GW)],
                        sem,
                    )
                )
            for c in copies:
                c.wait()

            def clamp_body(i, _):
                v = rows_v[i]
                rows_v[i] = jnp.minimum(jnp.maximum(v, 0.0), 1.0)
                return 0

            lax.fori_loop(0, CHUNK, clamp_body, 0)
            pltpu.sync_copy(rows_v, out_hbm.at[pl.ds(row_off * GW, CHUNK)])
            return 0

        lax.fori_loop(0, n_chunks, chunk_body, 0)

    return k(x2, W)


def kernel(x, W):
    b, h = x.shape
    n = b * h
    x2 = x.reshape(n // GW, GW).astype(jnp.int32)
    out = _sc_gather_clamp(x2, W, n // GW)
    return out.reshape(b, h, D)


# speedup vs baseline: 2.0848x; 1.0005x over previous
import functools

import jax
import jax.numpy as jnp
from jax import lax
from jax.experimental import pallas as pl
from jax.experimental.pallas import tpu as pltpu
from jax.experimental.pallas import tpu_sc as plsc

TERMS = 1000000
D = 16
NC = 2
NS = 16
NW = NC * NS
GW = 128
K = 16
CHUNK = K * GW


def _sc_gather_clamp(xf, W, n_rows):
    rows_per_w = n_rows // NW
    n_chunks = rows_per_w // K

    mesh = plsc.VectorSubcoreMesh(core_axis_name="c", subcore_axis_name="s")

    @functools.partial(
        pl.kernel,
        mesh=mesh,
        compiler_params=pltpu.CompilerParams(use_tc_tiling_on_sc=False),
        out_type=jax.ShapeDtypeStruct((n_rows * GW, D), jnp.float32),
        scratch_types=[
            pltpu.VMEM((CHUNK,), jnp.int32),
            pltpu.VMEM((CHUNK, D), jnp.float32),
            pltpu.SemaphoreType.DMA,
        ],
    )
    def k(x_hbm, w_hbm, out_hbm, idx_v, rows_v, sem):
        wid = lax.axis_index("s") * NC + lax.axis_index("c")
        row_base = wid * rows_per_w

        def chunk_body(ci, _):
            row_off = row_base + ci * K
            pltpu.sync_copy(x_hbm.at[pl.ds(row_off * GW, CHUNK)], idx_v)
            pltpu.async_copy(w_hbm.at[idx_v], rows_v, sem).wait()

            def clamp_body(i, _):
                v = rows_v[i]
                rows_v[i] = jnp.minimum(jnp.maximum(v, 0.0), 1.0)
                return 0

            lax.fori_loop(0, CHUNK, clamp_body, 0)
            pltpu.sync_copy(rows_v, out_hbm.at[pl.ds(row_off * GW, CHUNK)])
            return 0

        lax.fori_loop(0, n_chunks, chunk_body, 0)

    return k(xf, W)


def kernel(x, W):
    b, h = x.shape
    n = b * h
    xf = x.reshape(n).astype(jnp.int32)
    out = _sc_gather_clamp(xf, W, n // GW)
    return out.reshape(b, h, D)


# speedup vs baseline: 2.5088x; 1.2034x over previous
import functools

import jax
import jax.numpy as jnp
from jax import lax
from jax.experimental import pallas as pl
from jax.experimental.pallas import tpu as pltpu
from jax.experimental.pallas import tpu_sc as plsc

TERMS = 1000000
D = 16
NC = 2
NS = 16
NW = NC * NS
CHUNK = 2048
NBUF = 3


def _tc_clamp_table(W):
    wf = W.reshape(TERMS * D // 128, 128)
    rows = wf.shape[0]
    blk = 5000
    grid = rows // blk

    def body(w_ref, o_ref):
        o_ref[...] = jnp.clip(w_ref[...], 0.0, 1.0)

    out = pl.pallas_call(
        body,
        grid=(grid,),
        in_specs=[pl.BlockSpec((blk, 128), lambda i: (i, 0))],
        out_specs=pl.BlockSpec((blk, 128), lambda i: (i, 0)),
        out_shape=jax.ShapeDtypeStruct((rows, 128), jnp.float32),
    )(wf)
    return out.reshape(TERMS, D)


def _sc_gather(xf, Wc, n):
    rows_per_w = n // NW
    n_chunks = rows_per_w // CHUNK

    mesh = plsc.VectorSubcoreMesh(core_axis_name="c", subcore_axis_name="s")

    @functools.partial(
        pl.kernel,
        mesh=mesh,
        compiler_params=pltpu.CompilerParams(use_tc_tiling_on_sc=False),
        out_type=jax.ShapeDtypeStruct((n, D), jnp.float32),
        scratch_types=[
            pltpu.VMEM((NBUF, CHUNK), jnp.int32),
            pltpu.VMEM((NBUF, CHUNK, D), jnp.float32),
        ]
        + [pltpu.SemaphoreType.DMA] * (3 * NBUF),
    )
    def k(x_hbm, w_hbm, out_hbm, idx_v, rows_v, *sems):
        sem_i = sems[0:NBUF]
        sem_g = sems[NBUF:2 * NBUF]
        sem_o = sems[2 * NBUF:3 * NBUF]
        wid = lax.axis_index("s") * NC + lax.axis_index("c")
        base = wid * rows_per_w

        def fire_idx(ci, b):
            pltpu.async_copy(
                x_hbm.at[pl.ds(base + ci * CHUNK, CHUNK)],
                idx_v.at[b], sem_i[b])

        def fire_gather(ci, b):
            pltpu.async_copy(w_hbm.at[idx_v.at[b]], rows_v.at[b], sem_g[b])

        def fire_out(ci, b):
            pltpu.async_copy(
                rows_v.at[b],
                out_hbm.at[pl.ds(base + ci * CHUNK, CHUNK)], sem_o[b])

        def drain_idx(ci, b):
            pltpu.make_async_copy(
                x_hbm.at[pl.ds(base + ci * CHUNK, CHUNK)],
                idx_v.at[b], sem_i[b]).wait()

        def drain_gather(ci, b):
            pltpu.make_async_copy(
                w_hbm.at[idx_v.at[b]], rows_v.at[b], sem_g[b]).wait()

        def drain_out(ci, b):
            pltpu.make_async_copy(
                rows_v.at[b],
                out_hbm.at[pl.ds(base + ci * CHUNK, CHUNK)],
                sem_o[b]).wait()

        fire_idx(0, 0)

        def ring_body(c0):
            for u in range(NBUF):
                ci = c0 + u
                b = u
                nb = (u + 1) % NBUF
                pb = (u - 1) % NBUF

                @pl.when(ci + 1 < n_chunks)
                def _():
                    @pl.when(ci + 1 >= NBUF)
                    def _():
                        drain_out(ci + 1 - NBUF, nb)
                    fire_idx(ci + 1, nb)

                @pl.when(ci < n_chunks)
                def _():
                    drain_idx(ci, b)
                    fire_gather(ci, b)

                @pl.when((ci >= 1) & (ci <= n_chunks))
                def _():
                    drain_gather(ci - 1, pb)
                    fire_out(ci - 1, pb)

        pl.loop(0, n_chunks + NBUF, step=NBUF, unroll=False)(ring_body)

        for u in range(NBUF):
            ci = n_chunks - NBUF + u
            drain_out(ci, ci % NBUF)

    return k(xf, Wc)


def kernel(x, W):
    b, h = x.shape
    n = b * h
    xf = x.reshape(n).astype(jnp.int32)
    out = _sc_gather(xf, _tc_clamp_table(W), n)
    return out.reshape(b, h, D)


# speedup vs baseline: 5.6209x; 2.2405x over previous
import functools

import jax
import jax.numpy as jnp
from jax import lax
from jax.experimental import pallas as pl
from jax.experimental.pallas import tpu as pltpu
from jax.experimental.pallas import tpu_sc as plsc

TERMS = 1000000
D = 16
NC = 2
NS = 16
NW = NC * NS
CHUNK = 1024
NBUF = 3
BLK = 8192


def _tc_clamp_t(Wt):
    grid = (TERMS + BLK - 1) // BLK

    def body(w_ref, o_ref):
        o_ref[...] = jnp.clip(w_ref[...].T, 0.0, 1.0)

    return pl.pallas_call(
        body,
        grid=(grid,),
        in_specs=[pl.BlockSpec((D, BLK), lambda i: (0, i))],
        out_specs=pl.BlockSpec((BLK, D), lambda i: (i, 0)),
        out_shape=jax.ShapeDtypeStruct((TERMS, D), jnp.float32),
    )(Wt)


def _sc_gather_t(xf, Wc, n):
    rows_per_w = n // NW
    n_chunks = rows_per_w // CHUNK
    nh = n // 16384
    gpc = CHUNK // 128

    mesh = plsc.VectorSubcoreMesh(core_axis_name="c", subcore_axis_name="s")

    @functools.partial(
        pl.kernel,
        mesh=mesh,
        compiler_params=pltpu.CompilerParams(
            use_tc_tiling_on_sc=False, needs_layout_passes=False),
        out_type=jax.ShapeDtypeStruct((nh, 2, 128, 8, 128), jnp.float32),
        scratch_types=[
            pltpu.VMEM((NBUF, CHUNK), jnp.int32),
            pltpu.VMEM((NBUF, CHUNK, D), jnp.float32),
            pltpu.VMEM((NBUF, 2, gpc, 8, 128), jnp.float32),
        ]
        + [pltpu.SemaphoreType.DMA] * (3 * NBUF),
    )
    def k(x_hbm, w_hbm, out_hbm, idx_v, rows_v, t_v, *sems):
        sem_i = sems[0:NBUF]
        sem_g = sems[NBUF:2 * NBUF]
        sem_o = sems[2 * NBUF:3 * NBUF]
        wid = lax.axis_index("s") * NC + lax.axis_index("c")
        base = wid * rows_per_w
        lanes = lax.iota(jnp.int32, 16)

        def fire_idx(ci, b):
            pltpu.async_copy(
                x_hbm.at[pl.ds(base + ci * CHUNK, CHUNK)],
                idx_v.at[b], sem_i[b])

        def drain_idx(ci, b):
            pltpu.make_async_copy(
                x_hbm.at[pl.ds(base + ci * CHUNK, CHUNK)],
                idx_v.at[b], sem_i[b]).wait()

        def fire_gather(ci, b):
            pltpu.async_copy(w_hbm.at[idx_v.at[b]], rows_v.at[b], sem_g[b])

        def drain_gather(ci, b):
            pltpu.make_async_copy(
                w_hbm.at[idx_v.at[b]], rows_v.at[b], sem_g[b]).wait()

        def _out_slices(ci, b, dt):
            off = base + ci * CHUNK
            h = off >> 14
            bt0 = (off & 16383) >> 7
            return t_v.at[b, dt], out_hbm.at[h, dt, pl.ds(bt0, gpc)]

        def fire_out(ci, b):
            for dt in range(2):
                src, dst = _out_slices(ci, b, dt)
                pltpu.async_copy(src, dst, sem_o[b])

        def drain_out(ci, b):
            for dt in range(2):
                src, dst = _out_slices(ci, b, dt)
                pltpu.make_async_copy(src, dst, sem_o[b]).wait()

        def transpose_chunk(b):
            g_ref = rows_v.at[b]

            @plsc.parallel_loop(0, CHUNK, unroll=8)
            def _(i):
                bi0 = (i & 7) * 16
                di = (i >> 3) & 7
                g = (i >> 6) & (gpc - 1)
                dt = i >> 9
                rows = g * 128 + bi0 + lanes
                cols = jnp.full((16,), dt * 8 + di, jnp.int32)
                v = plsc.load_gather(g_ref, [rows, cols])
                t_v[b, dt, g, di, pl.ds(bi0, 16)] = v

        fire_idx(0, 0)

        def ring_body(c0):
            for u in range(NBUF):
                ci = c0 + u
                b = u
                nb = (u + 1) % NBUF
                pb = (u - 1) % NBUF

                @pl.when(ci + 1 < n_chunks)
                def _():
                    @pl.when(ci + 1 >= NBUF)
                    def _():
                        drain_out(ci + 1 - NBUF, nb)
                    fire_idx(ci + 1, nb)

                @pl.when(ci < n_chunks)
                def _():
                    drain_idx(ci, b)
                    fire_gather(ci, b)

                @pl.when((ci >= 1) & (ci <= n_chunks))
                def _():
                    drain_gather(ci - 1, pb)
                    transpose_chunk(pb)
                    fire_out(ci - 1, pb)

        pl.loop(0, n_chunks + NBUF, step=NBUF, unroll=False)(ring_body)

        for u in range(NBUF):
            ci = n_chunks - NBUF + u
            drain_out(ci, ci % NBUF)

    return k(xf, Wc)


def kernel(x, W):
    b, h = x.shape
    n = b * h
    xf = jnp.transpose(x).reshape(n).astype(jnp.int32)
    wc = _tc_clamp_t(jnp.transpose(W))
    p5 = _sc_gather_t(xf, wc, n)
    return jnp.transpose(p5, (2, 4, 0, 1, 3)).reshape(b, h, D)


# speedup vs baseline: 7.0590x; 1.2559x over previous
import functools

import jax
import jax.numpy as jnp
from jax import lax
from jax.experimental import pallas as pl
from jax.experimental.pallas import tpu as pltpu
from jax.experimental.pallas import tpu_sc as plsc

TERMS = 1000000
D = 16
NC = 2
NS = 16
NW = NC * NS
CHUNK = 1024
NBUF = 3
BLK = 8192


def _tc_clamp_t(Wt):
    grid = (TERMS + BLK - 1) // BLK

    def body(w_ref, o_ref):
        o_ref[...] = jnp.clip(w_ref[...].T, 0.0, 1.0)

    return pl.pallas_call(
        body,
        grid=(grid,),
        in_specs=[pl.BlockSpec((D, BLK), lambda i: (0, i))],
        out_specs=pl.BlockSpec((BLK, D), lambda i: (i, 0)),
        out_shape=jax.ShapeDtypeStruct((TERMS, D), jnp.float32),
    )(Wt)


def _sc_gather_t(xf, Wc, n):
    rows_per_w = n // NW
    n_chunks = rows_per_w // CHUNK
    nh = n // 16384
    gpc = CHUNK // 128

    mesh = plsc.VectorSubcoreMesh(core_axis_name="c", subcore_axis_name="s")

    @functools.partial(
        pl.kernel,
        mesh=mesh,
        compiler_params=pltpu.CompilerParams(
            use_tc_tiling_on_sc=False, needs_layout_passes=False),
        out_type=jax.ShapeDtypeStruct((nh, 2, 128, 8, 128), jnp.float32),
        scratch_types=[
            pltpu.VMEM((NBUF, CHUNK), jnp.int32),
            pltpu.VMEM((NBUF, CHUNK, D), jnp.float32),
            pltpu.VMEM((NBUF, 2, gpc, 8, 128), jnp.float32),
        ]
        + [pltpu.SemaphoreType.DMA] * (3 * NBUF),
    )
    def k(x_hbm, w_hbm, out_hbm, idx_v, rows_v, t_v, *sems):
        sem_i = sems[0:NBUF]
        sem_g = sems[NBUF:2 * NBUF]
        sem_o = sems[2 * NBUF:3 * NBUF]
        wid = lax.axis_index("s") * NC + lax.axis_index("c")
        base = wid * rows_per_w
        lanes = lax.iota(jnp.int32, 16)

        def fire_idx(ci, b):
            pltpu.async_copy(
                x_hbm.at[pl.ds(base + ci * CHUNK, CHUNK)],
                idx_v.at[b], sem_i[b])

        def drain_idx(ci, b):
            pltpu.make_async_copy(
                x_hbm.at[pl.ds(base + ci * CHUNK, CHUNK)],
                idx_v.at[b], sem_i[b]).wait()

        def fire_gather(ci, b):
            pltpu.async_copy(w_hbm.at[idx_v.at[b]], rows_v.at[b], sem_g[b])

        def drain_gather(ci, b):
            pltpu.make_async_copy(
                w_hbm.at[idx_v.at[b]], rows_v.at[b], sem_g[b]).wait()

        def _out_slices(ci, b, dt):
            off = base + ci * CHUNK
            h = off >> 14
            bt0 = (off & 16383) >> 7
            return t_v.at[b, dt], out_hbm.at[h, dt, pl.ds(bt0, gpc)]

        def fire_out(ci, b):
            for dt in range(2):
                src, dst = _out_slices(ci, b, dt)
                pltpu.async_copy(src, dst, sem_o[b])

        def drain_out(ci, b):
            for dt in range(2):
                src, dst = _out_slices(ci, b, dt)
                pltpu.make_async_copy(src, dst, sem_o[b]).wait()

        col_vecs = [jnp.full((16,), d, jnp.int32) for d in range(D)]

        def transpose_chunk(b):
            g_ref = rows_v.at[b]

            @plsc.parallel_loop(0, CHUNK // 16, unroll=2)
            def _(j):
                rows = (j << 4) + lanes
                g = j >> 3
                bi0 = (j & 7) << 4
                for d in range(D):
                    v = plsc.load_gather(g_ref, [rows, col_vecs[d]])
                    t_v[b, d >> 3, g, d & 7, pl.ds(bi0, 16)] = v

        fire_idx(0, 0)

        def ring_body(c0):
            for u in range(NBUF):
                ci = c0 + u
                b = u
                nb = (u + 1) % NBUF
                pb = (u - 1) % NBUF

                @pl.when(ci + 1 < n_chunks)
                def _():
                    @pl.when(ci + 1 >= NBUF)
                    def _():
                        drain_out(ci + 1 - NBUF, nb)
                    fire_idx(ci + 1, nb)

                @pl.when(ci < n_chunks)
                def _():
                    drain_idx(ci, b)
                    fire_gather(ci, b)

                @pl.when((ci >= 1) & (ci <= n_chunks))
                def _():
                    drain_gather(ci - 1, pb)
                    transpose_chunk(pb)
                    fire_out(ci - 1, pb)

        pl.loop(0, n_chunks + NBUF, step=NBUF, unroll=False)(ring_body)

        for u in range(NBUF):
            ci = n_chunks - NBUF + u
            drain_out(ci, ci % NBUF)

    return k(xf, Wc)


def kernel(x, W):
    b, h = x.shape
    n = b * h
    xf = jnp.transpose(x).reshape(n).astype(jnp.int32)
    wc = _tc_clamp_t(jnp.transpose(W))
    p5 = _sc_gather_t(xf, wc, n)
    return jnp.transpose(p5, (2, 4, 0, 1, 3)).reshape(b, h, D)


# speedup vs baseline: 7.5443x; 1.0687x over previous
import functools

import jax
import jax.numpy as jnp
from jax import lax
from jax.experimental import pallas as pl
from jax.experimental.pallas import tpu as pltpu
from jax.experimental.pallas import tpu_sc as plsc

TERMS = 1000000
D = 16
NC = 2
NS = 16
NW = NC * NS
CHUNK = 1024
NBUF = 3
BLK = 8192


def _tc_clamp_t(Wt):
    grid = (TERMS + BLK - 1) // BLK

    def body(w_ref, o_ref):
        o_ref[...] = jnp.clip(w_ref[...].T, 0.0, 1.0)

    return pl.pallas_call(
        body,
        grid=(grid,),
        in_specs=[pl.BlockSpec((D, BLK), lambda i: (0, i))],
        out_specs=pl.BlockSpec((BLK, D), lambda i: (i, 0)),
        out_shape=jax.ShapeDtypeStruct((TERMS, D), jnp.float32),
    )(Wt)


def _sc_gather_t(xf, Wc, n):
    rows_per_w = n // NW
    n_chunks = rows_per_w // CHUNK
    nh = n // 16384
    gpc = CHUNK // 128

    mesh = plsc.VectorSubcoreMesh(core_axis_name="c", subcore_axis_name="s")

    @functools.partial(
        pl.kernel,
        mesh=mesh,
        compiler_params=pltpu.CompilerParams(
            use_tc_tiling_on_sc=False, needs_layout_passes=False),
        out_type=jax.ShapeDtypeStruct((nh, 2, 128, 8, 128), jnp.float32),
        scratch_types=[
            pltpu.VMEM((NBUF, CHUNK), jnp.int32),
            pltpu.VMEM((NBUF, CHUNK, D), jnp.float32),
            pltpu.VMEM((NBUF, 2, gpc, 8, 128), jnp.float32),
        ]
        + [pltpu.SemaphoreType.DMA] * (3 * NBUF),
    )
    def k(x_hbm, w_hbm, out_hbm, idx_v, rows_v, t_v, *sems):
        sem_i = sems[0:NBUF]
        sem_g = sems[NBUF:2 * NBUF]
        sem_o = sems[2 * NBUF:3 * NBUF]
        wid = lax.axis_index("s") * NC + lax.axis_index("c")
        base = wid * rows_per_w
        lanes = lax.iota(jnp.int32, 16)

        def fire_idx(ci, b):
            pltpu.async_copy(
                x_hbm.at[pl.ds(base + ci * CHUNK, CHUNK)],
                idx_v.at[b], sem_i[b])

        def drain_idx(ci, b):
            pltpu.make_async_copy(
                x_hbm.at[pl.ds(base + ci * CHUNK, CHUNK)],
                idx_v.at[b], sem_i[b]).wait()

        def fire_gather(ci, b):
            pltpu.async_copy(w_hbm.at[idx_v.at[b]], rows_v.at[b], sem_g[b])

        def drain_gather(ci, b):
            pltpu.make_async_copy(
                w_hbm.at[idx_v.at[b]], rows_v.at[b], sem_g[b]).wait()

        def _out_slices(ci, b, dt):
            off = base + ci * CHUNK
            h = off >> 14
            bt0 = (off & 16383) >> 7
            return t_v.at[b, dt], out_hbm.at[h, dt, pl.ds(bt0, gpc)]

        def fire_out(ci, b):
            for dt in range(2):
                src, dst = _out_slices(ci, b, dt)
                pltpu.async_copy(src, dst, sem_o[b])

        def drain_out(ci, b):
            for dt in range(2):
                src, dst = _out_slices(ci, b, dt)
                pltpu.make_async_copy(src, dst, sem_o[b]).wait()

        col_vecs = [jnp.full((16,), d, jnp.int32) for d in range(D)]

        def transpose_chunk(b):
            g_ref = rows_v.at[b]

            @plsc.parallel_loop(0, CHUNK // 16, unroll=2)
            def _(j):
                rows = (j << 4) + lanes
                g = j >> 3
                bi0 = (j & 7) << 4
                for d in range(D):
                    v = plsc.load_gather(g_ref, [rows, col_vecs[d]])
                    t_v[b, d >> 3, g, d & 7, pl.ds(bi0, 16)] = v

        fire_idx(0, 0)
        fire_idx(1, 1)
        drain_idx(0, 0)
        fire_gather(0, 0)

        n_iters = n_chunks + 2
        assert n_iters % NBUF == 0

        def ring_body(c0):
            for u in range(NBUF):
                ci = c0 + u
                b0 = u
                b1 = (u + 1) % NBUF
                b2 = (u + 2) % NBUF

                @pl.when(ci + 2 < n_chunks)
                def _():
                    fire_idx(ci + 2, b2)

                @pl.when(ci + 1 < n_chunks)
                def _():
                    drain_idx(ci + 1, b1)
                    fire_gather(ci + 1, b1)

                @pl.when(ci >= 2)
                def _():
                    drain_out(ci - 2, b1)

                @pl.when(ci < n_chunks)
                def _():
                    drain_gather(ci, b0)
                    transpose_chunk(b0)
                    fire_out(ci, b0)

        pl.loop(0, n_iters, step=NBUF, unroll=False)(ring_body)

    return k(xf, Wc)


def kernel(x, W):
    b, h = x.shape
    n = b * h
    xf = jnp.transpose(x).reshape(n).astype(jnp.int32)
    wc = _tc_clamp_t(jnp.transpose(W))
    p5 = _sc_gather_t(xf, wc, n)
    return jnp.transpose(p5, (2, 4, 0, 1, 3)).reshape(b, h, D)
